# Initial kernel scaffold; baseline (speedup 1.0000x reference)
import jax
import jax.numpy as jnp
from jax.experimental import pallas as pl


def kernel(h, e, edge_index, snorm_n, snorm_e, W_embed, W_init, b_init, W_layers, b_layers, W_ro, W_pred, b_pred):
    raise NotImplementedError("write your pallas kernel here")


# speedup vs baseline: 4.6645x; 4.6645x over previous
import functools

import jax
import jax.numpy as jnp
from jax import lax
from jax.experimental import pallas as pl
from jax.experimental.pallas import tpu as pltpu
from jax.experimental.pallas import tpu_sc as plsc

N = 10000
E = 320000
DIN = 128
HID = 64

NC = 2
NS = 16
NW = NC * NS
CH = 128
CPW = 80
EPW = CPW * CH
EPAD = NW * EPW
NROWS = EPAD // CH
NPAD = 10240
RPS = NPAD // NS

_mesh = plsc.VectorSubcoreMesh(
    core_axis_name="c", subcore_axis_name="s", num_cores=NC, num_subcores=NS
)
_sc_params = pltpu.CompilerParams(use_tc_tiling_on_sc=False)


def _t1_body(h_ref, we_ref, wt_ref, b_ref, out_ref):
    hn = jnp.dot(h_ref[...], we_ref[...], preferred_element_type=jnp.float32)
    a = jnp.dot(hn, wt_ref[...], preferred_element_type=jnp.float32) + b_ref[...]
    rows = lax.broadcasted_iota(jnp.int32, (NPAD, HID), 0)
    out_ref[...] = jnp.where(rows < N, a, 0.0)


def _t2_body(p_ref, w_ref, b_ref, agg_ref, c_ref):
    agg = p_ref[0] + p_ref[1]
    agg_ref[...] = agg
    c = jnp.dot(agg, w_ref[...], preferred_element_type=jnp.float32) + b_ref[...]
    c_ref[...] = jnp.maximum(c, 0.0)


def _t3_body(a_ref, p_ref, w_ref, b_ref, agg_ref, c_ref):
    agg = a_ref[...] + p_ref[0] + p_ref[1]
    agg_ref[...] = agg
    c = jnp.dot(agg, w_ref[...], preferred_element_type=jnp.float32) + b_ref[...]
    c_ref[...] = jnp.maximum(c, 0.0)


def _t4_body(a_ref, p_ref, w_ref, b_ref, od_ref, cs_ref, c1_ref, c2_ref,
             wro_ref, wpred_ref, bpred_ref, out_ref):
    agg3 = a_ref[...] + p_ref[0] + p_ref[1]
    c3 = jnp.maximum(
        jnp.dot(agg3, w_ref[...], preferred_element_type=jnp.float32) + b_ref[...],
        0.0,
    )
    od = (od_ref[:N] + od_ref[NPAD:NPAD + N])[None, :]
    csum = c1_ref[:N] + c2_ref[:N] + c3[:N]
    tot = jnp.sum(cs_ref[...], axis=0)[None, :]
    tot = tot + jnp.dot(od, csum, preferred_element_type=jnp.float32)
    wf = jnp.dot(wro_ref[...], wpred_ref[...], preferred_element_type=jnp.float32)
    out_ref[...] = (
        jnp.dot(tot / N, wf, preferred_element_type=jnp.float32) + bpred_ref[...]
    )


def _tc_call(body, out_shapes):
    return pl.pallas_call(body, out_shape=out_shapes)


def _zero_node_acc(s, zb_ref, acc_ref):
    def zrow(i, _):
        for r in range(HID // 16):
            zb_ref[i, pl.ds(r * 16, 16)] = jnp.zeros((16,), jnp.float32)
        return 0
    lax.fori_loop(0, RPS, zrow, 0)
    pltpu.sync_copy(zb_ref, acc_ref.at[pl.ds(s * RPS, RPS)])


def _s1_body(apre_hbm, e_hbm, src_hbm, dst_hbm, wb_hbm,
             part_hbm, od_hbm, cs_hbm,
             srcs_v, dsts_v, e_v, rows_v, wb_v, ones_v, zb_v, zv_v, cs_v,
             acc_sh, od_sh):
    c = lax.axis_index("c")
    s = lax.axis_index("s")
    w = c * NS + s

    _zero_node_acc(s, zb_v, acc_sh)
    for r in range(RPS // 16):
        zv_v[pl.ds(r * 16, 16)] = jnp.zeros((16,), jnp.float32)
    pltpu.sync_copy(zv_v, od_sh.at[pl.ds(s * RPS, RPS)])
    for r in range(CH // 16):
        ones_v[pl.ds(r * 16, 16)] = jnp.ones((16,), jnp.float32)

    pltpu.sync_copy(src_hbm.at[pl.ds(w * CPW, CPW)], srcs_v)
    pltpu.sync_copy(dst_hbm.at[pl.ds(w * CPW, CPW)], dsts_v)
    pltpu.sync_copy(wb_hbm, wb_v)
    wb = [[wb_v[k, pl.ds(r * 16, 16)] for r in range(HID // 16)] for k in range(6)]

    plsc.subcore_barrier()

    ebase = w * EPW

    def chunk(j, carry):
        pltpu.sync_copy(e_hbm.at[pl.ds(ebase + j * CH, CH)], e_v)
        pltpu.sync_copy(apre_hbm.at[srcs_v.at[j]], rows_v)

        def edge(i, cs):
            ev = e_v[i, pl.ds(0, 16)]
            ek = [ev[k] for k in range(6)]
            new = []
            for r in range(HID // 16):
                v = rows_v[i, pl.ds(r * 16, 16)]
                for k in range(6):
                    v = v + ek[k] * wb[k][r]
                v = jnp.maximum(v, 0.0)
                rows_v[i, pl.ds(r * 16, 16)] = v
                new.append(cs[r] + v)
            return tuple(new)

        carry = lax.fori_loop(0, CH, edge, carry)
        pltpu.sync_copy(rows_v, acc_sh.at[dsts_v.at[j]], add=True)
        pltpu.sync_copy(ones_v, od_sh.at[srcs_v.at[j]], add=True)
        return carry

    zero16 = jnp.zeros((16,), jnp.float32)
    csum = lax.fori_loop(0, CPW, chunk, (zero16,) * (HID // 16))

    for r in range(HID // 16):
        cs_v[pl.ds(r * 16, 16)] = csum[r]
    pltpu.sync_copy(cs_v, cs_hbm.at[w])

    plsc.subcore_barrier()
    pltpu.sync_copy(acc_sh.at[pl.ds(s * RPS, RPS)],
                    part_hbm.at[c, pl.ds(s * RPS, RPS)])
    pltpu.sync_copy(od_sh.at[pl.ds(s * RPS, RPS)],
                    od_hbm.at[pl.ds(c * NPAD + s * RPS, RPS)])


def _s2_body(tab_hbm, src_hbm, dst_hbm, part_hbm,
             srcs_v, dsts_v, rows_v, zb_v, acc_sh):
    c = lax.axis_index("c")
    s = lax.axis_index("s")
    w = c * NS + s

    _zero_node_acc(s, zb_v, acc_sh)
    pltpu.sync_copy(src_hbm.at[pl.ds(w * CPW, CPW)], srcs_v)
    pltpu.sync_copy(dst_hbm.at[pl.ds(w * CPW, CPW)], dsts_v)
    plsc.subcore_barrier()

    def chunk(j, _):
        pltpu.sync_copy(tab_hbm.at[srcs_v.at[j]], rows_v)
        pltpu.sync_copy(rows_v, acc_sh.at[dsts_v.at[j]], add=True)
        return 0

    lax.fori_loop(0, CPW, chunk, 0)

    plsc.subcore_barrier()
    pltpu.sync_copy(acc_sh.at[pl.ds(s * RPS, RPS)],
                    part_hbm.at[c, pl.ds(s * RPS, RPS)])


_s1_call = functools.partial(
    pl.kernel,
    out_type=(
        jax.ShapeDtypeStruct((NC, NPAD, HID), jnp.float32),
        jax.ShapeDtypeStruct((NC * NPAD,), jnp.float32),
        jax.ShapeDtypeStruct((NW, HID), jnp.float32),
    ),
    mesh=_mesh,
    scratch_types=[
        pltpu.VMEM((CPW, CH), jnp.int32),
        pltpu.VMEM((CPW, CH), jnp.int32),
        pltpu.VMEM((CH, 16), jnp.float32),
        pltpu.VMEM((CH, HID), jnp.float32),
        pltpu.VMEM((8, HID), jnp.float32),
        pltpu.VMEM((CH,), jnp.float32),
        pltpu.VMEM((RPS, HID), jnp.float32),
        pltpu.VMEM((RPS,), jnp.float32),
        pltpu.VMEM((HID,), jnp.float32),
        pltpu.VMEM_SHARED((NPAD, HID), jnp.float32),
        pltpu.VMEM_SHARED((NPAD,), jnp.float32),
    ],
    compiler_params=_sc_params,
)(_s1_body)

_s2_call = functools.partial(
    pl.kernel,
    out_type=jax.ShapeDtypeStruct((NC, NPAD, HID), jnp.float32),
    mesh=_mesh,
    scratch_types=[
        pltpu.VMEM((CPW, CH), jnp.int32),
        pltpu.VMEM((CPW, CH), jnp.int32),
        pltpu.VMEM((CH, HID), jnp.float32),
        pltpu.VMEM((RPS, HID), jnp.float32),
        pltpu.VMEM_SHARED((NPAD, HID), jnp.float32),
    ],
    compiler_params=_sc_params,
)(_s2_body)


def kernel(h, e, edge_index, snorm_n, snorm_e, W_embed, W_init, b_init,
           W_layers, b_layers, W_ro, W_pred, b_pred):
    f32 = jnp.float32
    h_pad = jnp.pad(h.astype(f32), ((0, NPAD - N), (0, 0)))
    e_pad = jnp.pad(e.astype(f32), ((0, EPAD - E), (0, 16 - e.shape[1])))
    src = jnp.pad(edge_index[0], (0, EPAD - E), constant_values=N).reshape(NROWS, CH)
    dst = jnp.pad(edge_index[1], (0, EPAD - E), constant_values=N).reshape(NROWS, CH)
    wtop = W_init[:HID]
    wbot = jnp.pad(W_init[HID:HID + 6], ((0, 2), (0, 0)))

    apre = _tc_call(_t1_body, jax.ShapeDtypeStruct((NPAD, HID), f32))(
        h_pad, W_embed, wtop, b_init[None, :]
    )

    p0, od, cs = _s1_call(apre, e_pad, src, dst, wbot)

    node_sh = jax.ShapeDtypeStruct((NPAD, HID), f32)
    agg1, c1 = _tc_call(_t2_body, (node_sh, node_sh))(
        p0, W_layers[0], b_layers[0][None, :]
    )
    p1 = _s2_call(c1, src, dst)
    agg2, c2 = _tc_call(_t3_body, (node_sh, node_sh))(
        agg1, p1, W_layers[1], b_layers[1][None, :]
    )
    p2 = _s2_call(c2, src, dst)
    out = _tc_call(_t4_body, jax.ShapeDtypeStruct((1, 1), f32))(
        agg2, p2, W_layers[2], b_layers[2][None, :], od, cs, c1, c2,
        W_ro, W_pred, b_pred[None, :]
    )
    return out


# speedup vs baseline: 9.2809x; 1.9897x over previous
import functools

import jax
import jax.numpy as jnp
from jax import lax
from jax.experimental import pallas as pl
from jax.experimental.pallas import tpu as pltpu
from jax.experimental.pallas import tpu_sc as plsc

N = 10000
E = 320000
DIN = 128
HID = 64

NC = 2
NS = 16
NW = NC * NS
CH = 128
CPW = 80
EPW = CPW * CH
EPAD = NW * EPW
NROWS = EPAD // CH
NPAD = 10240
RPS = NPAD // NS

_mesh = plsc.VectorSubcoreMesh(
    core_axis_name="c", subcore_axis_name="s", num_cores=NC, num_subcores=NS
)
_sc_params = pltpu.CompilerParams(use_tc_tiling_on_sc=False)


def _t1_body(h_ref, we_ref, wt_ref, b_ref, out_ref):
    hn = jnp.dot(h_ref[...], we_ref[...], preferred_element_type=jnp.float32)
    a = jnp.dot(hn, wt_ref[...], preferred_element_type=jnp.float32) + b_ref[...]
    rows = lax.broadcasted_iota(jnp.int32, (NPAD, HID), 0)
    out_ref[...] = jnp.where(rows < N, a, 0.0)


def _t2_body(p_ref, w_ref, b_ref, agg_ref, c_ref, cs_ref):
    agg = p_ref[0] + p_ref[1]
    agg_ref[...] = agg
    c = jnp.dot(agg, w_ref[...], preferred_element_type=jnp.float32) + b_ref[...]
    c_ref[...] = jnp.maximum(c, 0.0)
    cs_ref[...] = jnp.sum(agg, axis=0)[None, :]


def _t3_body(a_ref, p_ref, w_ref, b_ref, agg_ref, c_ref):
    agg = a_ref[...] + p_ref[0] + p_ref[1]
    agg_ref[...] = agg
    c = jnp.dot(agg, w_ref[...], preferred_element_type=jnp.float32) + b_ref[...]
    c_ref[...] = jnp.maximum(c, 0.0)


def _t4_body(a_ref, p_ref, w_ref, b_ref, od_ref, cs_ref, c1_ref, c2_ref,
             wro_ref, wpred_ref, bpred_ref, out_ref):
    agg3 = a_ref[...] + p_ref[0] + p_ref[1]
    c3 = jnp.maximum(
        jnp.dot(agg3, w_ref[...], preferred_element_type=jnp.float32) + b_ref[...],
        0.0,
    )
    od = (od_ref[:N] + od_ref[NPAD:NPAD + N])[None, :]
    csum = c1_ref[:N] + c2_ref[:N] + c3[:N]
    tot = cs_ref[...]
    tot = tot + jnp.dot(od, csum, preferred_element_type=jnp.float32)
    wf = jnp.dot(wro_ref[...], wpred_ref[...], preferred_element_type=jnp.float32)
    out_ref[...] = (
        jnp.dot(tot / N, wf, preferred_element_type=jnp.float32) + bpred_ref[...]
    )


def _tc_call(body, out_shapes):
    return pl.pallas_call(body, out_shape=out_shapes)


NB = 4
AHEAD = 2
NG = CPW // NB


def _zero_node_acc(s, zb_ref, acc_ref):
    def zrow(i, _):
        for r in range(HID // 16):
            zb_ref[i, pl.ds(r * 16, 16)] = jnp.zeros((16,), jnp.float32)
        return 0
    lax.fori_loop(0, CH, zrow, 0)
    for k in range(RPS // CH):
        pltpu.sync_copy(zb_ref, acc_ref.at[pl.ds(s * RPS + k * CH, CH)])


def _s1_body(apre_hbm, e_hbm, src_hbm, dst_hbm, wb_hbm,
             part_hbm, od_hbm,
             srcs_v, dsts_v, ebuf_v, rows_v, wb_v, ones_v, zb_v, zv_v,
             acc_sh, od_sh, gsem, esem, ssem, osem):
    c = lax.axis_index("c")
    s = lax.axis_index("s")
    w = c * NS + s

    _zero_node_acc(s, zb_v, acc_sh)
    for r in range(RPS // 16):
        zv_v[pl.ds(r * 16, 16)] = jnp.zeros((16,), jnp.float32)
    pltpu.sync_copy(zv_v, od_sh.at[pl.ds(s * RPS, RPS)])
    for r in range(CH // 16):
        ones_v[pl.ds(r * 16, 16)] = jnp.ones((16,), jnp.float32)

    pltpu.sync_copy(src_hbm.at[pl.ds(w * CPW, CPW)], srcs_v)
    pltpu.sync_copy(dst_hbm.at[pl.ds(w * CPW, CPW)], dsts_v)
    pltpu.sync_copy(wb_hbm, wb_v)
    wb = [[wb_v[k, pl.ds(r * 16, 16)] for r in range(HID // 16)] for k in range(6)]

    plsc.subcore_barrier()

    ebase = w * EPW

    def g_start(t, b):
        pltpu.make_async_copy(apre_hbm.at[srcs_v.at[t]], rows_v.at[b],
                              gsem.at[b]).start()
        pltpu.make_async_copy(e_hbm.at[pl.ds(ebase + t * CH, CH)],
                              ebuf_v.at[b], esem.at[b]).start()

    def g_wait(t, b):
        pltpu.make_async_copy(apre_hbm.at[srcs_v.at[t]], rows_v.at[b],
                              gsem.at[b]).wait()
        pltpu.make_async_copy(e_hbm.at[pl.ds(ebase + t * CH, CH)],
                              ebuf_v.at[b], esem.at[b]).wait()

    def s_start(t, b):
        pltpu.make_async_copy(rows_v.at[b], acc_sh.at[dsts_v.at[t]],
                              ssem.at[b]).start(add=True)
        pltpu.make_async_copy(ones_v, od_sh.at[srcs_v.at[t]], osem).start(add=True)

    def s_wait(t, b):
        pltpu.make_async_copy(rows_v.at[b], acc_sh.at[dsts_v.at[t]],
                              ssem.at[b]).wait()

    def compute(b):
        rows_b = rows_v.at[b]
        ebuf_b = ebuf_v.at[b]

        def edge(i, _):
            ve = ebuf_b[i, pl.ds(0, 16)]
            ek = [ve[k] for k in range(6)]
            for r in range(HID // 16):
                v = rows_b[i, pl.ds(r * 16, 16)]
                for k in range(6):
                    v = v + ek[k] * wb[k][r]
                rows_b[i, pl.ds(r * 16, 16)] = jnp.maximum(v, 0.0)
            return 0

        lax.fori_loop(0, CH, edge, 0)

    for t in range(AHEAD):
        g_start(t, t)

    def outer(g, _):
        for b in range(NB):
            j = g * NB + b
            g_wait(j, b)
            compute(b)
            s_start(j, b)
            t = j + AHEAD
            bt = (b + AHEAD) % NB
            if b < AHEAD:
                @pl.when(g >= 1)
                def _():
                    s_wait(j - (NB - AHEAD), bt)
                g_start(t, bt)
            else:
                @pl.when(g <= NG - 2)
                def _():
                    s_wait(j - (NB - AHEAD), bt)
                    g_start(t, bt)
        return 0

    lax.fori_loop(0, NG, outer, 0)

    for i in range(NB):
        t = CPW - NB + i
        s_wait(t, t % NB)

    def odrain(j, _):
        pltpu.make_async_copy(ones_v, od_sh.at[srcs_v.at[j]], osem).wait()
        return 0
    lax.fori_loop(0, CPW, odrain, 0)

    plsc.subcore_barrier()
    pltpu.sync_copy(acc_sh.at[pl.ds(s * RPS, RPS)],
                    part_hbm.at[c, pl.ds(s * RPS, RPS)])
    pltpu.sync_copy(od_sh.at[pl.ds(s * RPS, RPS)],
                    od_hbm.at[pl.ds(c * NPAD + s * RPS, RPS)])


def _s2_body(tab_hbm, src_hbm, dst_hbm, part_hbm,
             srcs_v, dsts_v, rows_v, zb_v, acc_sh, gsem, ssem):
    c = lax.axis_index("c")
    s = lax.axis_index("s")
    w = c * NS + s

    _zero_node_acc(s, zb_v, acc_sh)
    pltpu.sync_copy(src_hbm.at[pl.ds(w * CPW, CPW)], srcs_v)
    pltpu.sync_copy(dst_hbm.at[pl.ds(w * CPW, CPW)], dsts_v)
    plsc.subcore_barrier()

    def g_start(t, b):
        pltpu.make_async_copy(tab_hbm.at[srcs_v.at[t]], rows_v.at[b],
                              gsem.at[b]).start()

    def g_wait(t, b):
        pltpu.make_async_copy(tab_hbm.at[srcs_v.at[t]], rows_v.at[b],
                              gsem.at[b]).wait()

    def s_start(t, b):
        pltpu.make_async_copy(rows_v.at[b], acc_sh.at[dsts_v.at[t]],
                              ssem.at[b]).start(add=True)

    def s_wait(t, b):
        pltpu.make_async_copy(rows_v.at[b], acc_sh.at[dsts_v.at[t]],
                              ssem.at[b]).wait()

    for t in range(AHEAD):
        g_start(t, t)

    def outer(g, _):
        for b in range(NB):
            j = g * NB + b
            g_wait(j, b)
            s_start(j, b)
            t = j + AHEAD
            bt = (b + AHEAD) % NB
            if b < AHEAD:
                @pl.when(g >= 1)
                def _():
                    s_wait(j - (NB - AHEAD), bt)
                g_start(t, bt)
            else:
                @pl.when(g <= NG - 2)
                def _():
                    s_wait(j - (NB - AHEAD), bt)
                    g_start(t, bt)
        return 0

    lax.fori_loop(0, NG, outer, 0)

    for i in range(NB):
        t = CPW - NB + i
        s_wait(t, t % NB)

    plsc.subcore_barrier()
    pltpu.sync_copy(acc_sh.at[pl.ds(s * RPS, RPS)],
                    part_hbm.at[c, pl.ds(s * RPS, RPS)])


_s1_call = functools.partial(
    pl.kernel,
    out_type=(
        jax.ShapeDtypeStruct((NC, NPAD, HID), jnp.float32),
        jax.ShapeDtypeStruct((NC * NPAD,), jnp.float32),
    ),
    mesh=_mesh,
    scratch_types=[
        pltpu.VMEM((CPW, CH), jnp.int32),
        pltpu.VMEM((CPW, CH), jnp.int32),
        pltpu.VMEM((NB, CH, 16), jnp.float32),
        pltpu.VMEM((NB, CH, HID), jnp.float32),
        pltpu.VMEM((8, HID), jnp.float32),
        pltpu.VMEM((CH,), jnp.float32),
        pltpu.VMEM((CH, HID), jnp.float32),
        pltpu.VMEM((RPS,), jnp.float32),
        pltpu.VMEM_SHARED((NPAD, HID), jnp.float32),
        pltpu.VMEM_SHARED((NPAD,), jnp.float32),
        pltpu.SemaphoreType.DMA((NB,)),
        pltpu.SemaphoreType.DMA((NB,)),
        pltpu.SemaphoreType.DMA((NB,)),
        pltpu.SemaphoreType.DMA,
    ],
    compiler_params=_sc_params,
)(_s1_body)

_s2_call = functools.partial(
    pl.kernel,
    out_type=jax.ShapeDtypeStruct((NC, NPAD, HID), jnp.float32),
    mesh=_mesh,
    scratch_types=[
        pltpu.VMEM((CPW, CH), jnp.int32),
        pltpu.VMEM((CPW, CH), jnp.int32),
        pltpu.VMEM((NB, CH, HID), jnp.float32),
        pltpu.VMEM((CH, HID), jnp.float32),
        pltpu.VMEM_SHARED((NPAD, HID), jnp.float32),
        pltpu.SemaphoreType.DMA((NB,)),
        pltpu.SemaphoreType.DMA((NB,)),
    ],
    compiler_params=_sc_params,
)(_s2_body)


def kernel(h, e, edge_index, snorm_n, snorm_e, W_embed, W_init, b_init,
           W_layers, b_layers, W_ro, W_pred, b_pred):
    f32 = jnp.float32
    h_pad = jnp.pad(h.astype(f32), ((0, NPAD - N), (0, 0)))
    e_pad = jnp.pad(e.astype(f32), ((0, EPAD - E), (0, 16 - e.shape[1])))
    pad_idx = N + jnp.arange(EPAD - E, dtype=jnp.int32) % (NPAD - N)
    src = jnp.concatenate([edge_index[0], pad_idx]).reshape(NROWS, CH)
    dst = jnp.concatenate([edge_index[1], pad_idx]).reshape(NROWS, CH)
    wtop = W_init[:HID]
    wbot = jnp.pad(W_init[HID:HID + 6], ((0, 2), (0, 0)))

    apre = _tc_call(_t1_body, jax.ShapeDtypeStruct((NPAD, HID), f32))(
        h_pad, W_embed, wtop, b_init[None, :]
    )

    p0, od = _s1_call(apre, e_pad, src, dst, wbot)

    node_sh = jax.ShapeDtypeStruct((NPAD, HID), f32)
    agg1, c1, cs = _tc_call(_t2_body, (node_sh, node_sh,
                                       jax.ShapeDtypeStruct((1, HID), f32)))(
        p0, W_layers[0], b_layers[0][None, :]
    )
    p1 = _s2_call(c1, src, dst)
    agg2, c2 = _tc_call(_t3_body, (node_sh, node_sh))(
        agg1, p1, W_layers[1], b_layers[1][None, :]
    )
    p2 = _s2_call(c2, src, dst)
    out = _tc_call(_t4_body, jax.ShapeDtypeStruct((1, 1), f32))(
        agg2, p2, W_layers[2], b_layers[2][None, :], od, cs, c1, c2,
        W_ro, W_pred, b_pred[None, :]
    )
    return out


# speedup vs baseline: 9.4534x; 1.0186x over previous
import functools

import jax
import jax.numpy as jnp
from jax import lax
from jax.experimental import pallas as pl
from jax.experimental.pallas import tpu as pltpu
from jax.experimental.pallas import tpu_sc as plsc

N = 10000
E = 320000
DIN = 128
HID = 64

NC = 2
NS = 16
NW = NC * NS
CH = 128
CPW = 80
EPW = CPW * CH
EPAD = NW * EPW
NROWS = EPAD // CH
NPAD = 10240
RPS = NPAD // NS

_mesh = plsc.VectorSubcoreMesh(
    core_axis_name="c", subcore_axis_name="s", num_cores=NC, num_subcores=NS
)
_sc_params = pltpu.CompilerParams(use_tc_tiling_on_sc=False)


def _t1_body(h_ref, we_ref, wt_ref, b_ref, out_ref):
    hn = jnp.dot(h_ref[...], we_ref[...], preferred_element_type=jnp.float32)
    a = jnp.dot(hn, wt_ref[...], preferred_element_type=jnp.float32) + b_ref[...]
    rows = lax.broadcasted_iota(jnp.int32, (NPAD, HID), 0)
    out_ref[...] = jnp.where(rows < N, a, 0.0)


def _t2_body(p_ref, w_ref, b_ref, agg_ref, c_ref, cs_ref):
    agg = p_ref[0] + p_ref[1]
    agg_ref[...] = agg
    c = jnp.dot(agg, w_ref[...], preferred_element_type=jnp.float32) + b_ref[...]
    c_ref[...] = jnp.maximum(c, 0.0)
    cs_ref[...] = jnp.sum(agg, axis=0)[None, :]


def _t3_body(a_ref, p_ref, w_ref, b_ref, agg_ref, c_ref):
    agg = a_ref[...] + p_ref[0] + p_ref[1]
    agg_ref[...] = agg
    c = jnp.dot(agg, w_ref[...], preferred_element_type=jnp.float32) + b_ref[...]
    c_ref[...] = jnp.maximum(c, 0.0)


def _t4_body(a_ref, p_ref, w_ref, b_ref, od_ref, cs_ref, c1_ref, c2_ref,
             wro_ref, wpred_ref, bpred_ref, out_ref):
    agg3 = a_ref[...] + p_ref[0] + p_ref[1]
    c3 = jnp.maximum(
        jnp.dot(agg3, w_ref[...], preferred_element_type=jnp.float32) + b_ref[...],
        0.0,
    )
    od = (od_ref[:N] + od_ref[NPAD:NPAD + N])[None, :]
    csum = c1_ref[:N] + c2_ref[:N] + c3[:N]
    tot = cs_ref[...]
    tot = tot + jnp.dot(od, csum, preferred_element_type=jnp.float32)
    wf = jnp.dot(wro_ref[...], wpred_ref[...], preferred_element_type=jnp.float32)
    out_ref[...] = (
        jnp.dot(tot / N, wf, preferred_element_type=jnp.float32) + bpred_ref[...]
    )


def _tc_call(body, out_shapes):
    return pl.pallas_call(body, out_shape=out_shapes)


NB = 4
AHEAD = 2
NG = CPW // NB


def _zero_node_acc(s, zb_ref, acc_ref):
    def zrow(i, _):
        for r in range(HID // 16):
            zb_ref[i, pl.ds(r * 16, 16)] = jnp.zeros((16,), jnp.float32)
        return 0
    lax.fori_loop(0, CH, zrow, 0)
    for k in range(RPS // CH):
        pltpu.sync_copy(zb_ref, acc_ref.at[pl.ds(s * RPS + k * CH, CH)])


def _s1_body(apre_hbm, e_hbm, src_hbm, dst_hbm, wb_hbm,
             part_hbm, od_hbm,
             srcs_v, dsts_v, ebuf_v, rows_v, wb_v, ones_v, zb_v, zv_v,
             acc_sh, od_sh, gsem, esem, ssem, osem):
    c = lax.axis_index("c")
    s = lax.axis_index("s")
    w = c * NS + s

    _zero_node_acc(s, zb_v, acc_sh)
    for r in range(RPS // 16):
        zv_v[pl.ds(r * 16, 16)] = jnp.zeros((16,), jnp.float32)
    pltpu.sync_copy(zv_v, od_sh.at[pl.ds(s * RPS, RPS)])
    for r in range(CH // 16):
        ones_v[pl.ds(r * 16, 16)] = jnp.ones((16,), jnp.float32)

    pltpu.sync_copy(src_hbm.at[pl.ds(w * CPW, CPW)], srcs_v)
    pltpu.sync_copy(dst_hbm.at[pl.ds(w * CPW, CPW)], dsts_v)
    pltpu.sync_copy(wb_hbm, wb_v)
    wb = [[wb_v[k, pl.ds(r * 16, 16)] for r in range(HID // 16)] for k in range(6)]

    plsc.subcore_barrier()

    ebase = w * EPW // 2

    def g_start(t, b):
        pltpu.make_async_copy(apre_hbm.at[srcs_v.at[t]], rows_v.at[b],
                              gsem.at[b]).start()
        pltpu.make_async_copy(e_hbm.at[pl.ds(ebase + t * (CH // 2), CH // 2)],
                              ebuf_v.at[b], esem.at[b]).start()

    def g_wait(t, b):
        pltpu.make_async_copy(apre_hbm.at[srcs_v.at[t]], rows_v.at[b],
                              gsem.at[b]).wait()
        pltpu.make_async_copy(e_hbm.at[pl.ds(ebase + t * (CH // 2), CH // 2)],
                              ebuf_v.at[b], esem.at[b]).wait()

    def s_start(t, b):
        pltpu.make_async_copy(rows_v.at[b], acc_sh.at[dsts_v.at[t]],
                              ssem.at[b]).start(add=True)
        pltpu.make_async_copy(ones_v, od_sh.at[srcs_v.at[t]], osem).start(add=True)

    def s_wait(t, b):
        pltpu.make_async_copy(rows_v.at[b], acc_sh.at[dsts_v.at[t]],
                              ssem.at[b]).wait()

    def compute(b):
        rows_b = rows_v.at[b]
        ebuf_b = ebuf_v.at[b]

        def pair(i2, _):
            ve = ebuf_b[i2, pl.ds(0, 16)]
            for half in range(2):
                i = 2 * i2 + half
                ek = [ve[8 * half + k] for k in range(6)]
                for r in range(HID // 16):
                    v = rows_b[i, pl.ds(r * 16, 16)]
                    t01 = ek[0] * wb[0][r] + ek[1] * wb[1][r]
                    t23 = ek[2] * wb[2][r] + ek[3] * wb[3][r]
                    t45 = ek[4] * wb[4][r] + ek[5] * wb[5][r]
                    v = (v + t01) + (t23 + t45)
                    rows_b[i, pl.ds(r * 16, 16)] = jnp.maximum(v, 0.0)
            return 0

        lax.fori_loop(0, CH // 2, pair, 0)

    for t in range(AHEAD):
        g_start(t, t)

    def outer(g, _):
        for b in range(NB):
            j = g * NB + b
            g_wait(j, b)
            compute(b)
            s_start(j, b)
            t = j + AHEAD
            bt = (b + AHEAD) % NB
            if b < AHEAD:
                @pl.when(g >= 1)
                def _():
                    s_wait(j - (NB - AHEAD), bt)
                g_start(t, bt)
            else:
                @pl.when(g <= NG - 2)
                def _():
                    s_wait(j - (NB - AHEAD), bt)
                    g_start(t, bt)
        return 0

    lax.fori_loop(0, NG, outer, 0)

    for i in range(NB):
        t = CPW - NB + i
        s_wait(t, t % NB)

    def odrain(j, _):
        pltpu.make_async_copy(ones_v, od_sh.at[srcs_v.at[j]], osem).wait()
        return 0
    lax.fori_loop(0, CPW, odrain, 0)

    plsc.subcore_barrier()
    pltpu.sync_copy(acc_sh.at[pl.ds(s * RPS, RPS)],
                    part_hbm.at[c, pl.ds(s * RPS, RPS)])
    pltpu.sync_copy(od_sh.at[pl.ds(s * RPS, RPS)],
                    od_hbm.at[pl.ds(c * NPAD + s * RPS, RPS)])


def _s2_body(tab_hbm, src_hbm, dst_hbm, part_hbm,
             srcs_v, dsts_v, rows_v, zb_v, acc_sh, gsem, ssem):
    c = lax.axis_index("c")
    s = lax.axis_index("s")
    w = c * NS + s

    _zero_node_acc(s, zb_v, acc_sh)
    pltpu.sync_copy(src_hbm.at[pl.ds(w * CPW, CPW)], srcs_v)
    pltpu.sync_copy(dst_hbm.at[pl.ds(w * CPW, CPW)], dsts_v)
    plsc.subcore_barrier()

    def g_start(t, b):
        pltpu.make_async_copy(tab_hbm.at[srcs_v.at[t]], rows_v.at[b],
                              gsem.at[b]).start()

    def g_wait(t, b):
        pltpu.make_async_copy(tab_hbm.at[srcs_v.at[t]], rows_v.at[b],
                              gsem.at[b]).wait()

    def s_start(t, b):
        pltpu.make_async_copy(rows_v.at[b], acc_sh.at[dsts_v.at[t]],
                              ssem.at[b]).start(add=True)

    def s_wait(t, b):
        pltpu.make_async_copy(rows_v.at[b], acc_sh.at[dsts_v.at[t]],
                              ssem.at[b]).wait()

    for t in range(AHEAD):
        g_start(t, t)

    def outer(g, _):
        for b in range(NB):
            j = g * NB + b
            g_wait(j, b)
            s_start(j, b)
            t = j + AHEAD
            bt = (b + AHEAD) % NB
            if b < AHEAD:
                @pl.when(g >= 1)
                def _():
                    s_wait(j - (NB - AHEAD), bt)
                g_start(t, bt)
            else:
                @pl.when(g <= NG - 2)
                def _():
                    s_wait(j - (NB - AHEAD), bt)
                    g_start(t, bt)
        return 0

    lax.fori_loop(0, NG, outer, 0)

    for i in range(NB):
        t = CPW - NB + i
        s_wait(t, t % NB)

    plsc.subcore_barrier()
    pltpu.sync_copy(acc_sh.at[pl.ds(s * RPS, RPS)],
                    part_hbm.at[c, pl.ds(s * RPS, RPS)])


_s1_call = functools.partial(
    pl.kernel,
    out_type=(
        jax.ShapeDtypeStruct((NC, NPAD, HID), jnp.float32),
        jax.ShapeDtypeStruct((NC * NPAD,), jnp.float32),
    ),
    mesh=_mesh,
    scratch_types=[
        pltpu.VMEM((CPW, CH), jnp.int32),
        pltpu.VMEM((CPW, CH), jnp.int32),
        pltpu.VMEM((NB, CH // 2, 16), jnp.float32),
        pltpu.VMEM((NB, CH, HID), jnp.float32),
        pltpu.VMEM((8, HID), jnp.float32),
        pltpu.VMEM((CH,), jnp.float32),
        pltpu.VMEM((CH, HID), jnp.float32),
        pltpu.VMEM((RPS,), jnp.float32),
        pltpu.VMEM_SHARED((NPAD, HID), jnp.float32),
        pltpu.VMEM_SHARED((NPAD,), jnp.float32),
        pltpu.SemaphoreType.DMA((NB,)),
        pltpu.SemaphoreType.DMA((NB,)),
        pltpu.SemaphoreType.DMA((NB,)),
        pltpu.SemaphoreType.DMA,
    ],
    compiler_params=_sc_params,
)(_s1_body)

_s2_call = functools.partial(
    pl.kernel,
    out_type=jax.ShapeDtypeStruct((NC, NPAD, HID), jnp.float32),
    mesh=_mesh,
    scratch_types=[
        pltpu.VMEM((CPW, CH), jnp.int32),
        pltpu.VMEM((CPW, CH), jnp.int32),
        pltpu.VMEM((NB, CH, HID), jnp.float32),
        pltpu.VMEM((CH, HID), jnp.float32),
        pltpu.VMEM_SHARED((NPAD, HID), jnp.float32),
        pltpu.SemaphoreType.DMA((NB,)),
        pltpu.SemaphoreType.DMA((NB,)),
    ],
    compiler_params=_sc_params,
)(_s2_body)


def kernel(h, e, edge_index, snorm_n, snorm_e, W_embed, W_init, b_init,
           W_layers, b_layers, W_ro, W_pred, b_pred):
    f32 = jnp.float32
    h_pad = jnp.pad(h.astype(f32), ((0, NPAD - N), (0, 0)))
    e_pad = jnp.pad(e.astype(f32),
                    ((0, EPAD - E), (0, 8 - e.shape[1]))).reshape(EPAD // 2, 16)
    pad_idx = N + jnp.arange(EPAD - E, dtype=jnp.int32) % (NPAD - N)
    src = jnp.concatenate([edge_index[0], pad_idx]).reshape(NROWS, CH)
    dst = jnp.concatenate([edge_index[1], pad_idx]).reshape(NROWS, CH)
    wtop = W_init[:HID]
    wbot = jnp.pad(W_init[HID:HID + 6], ((0, 2), (0, 0)))

    apre = _tc_call(_t1_body, jax.ShapeDtypeStruct((NPAD, HID), f32))(
        h_pad, W_embed, wtop, b_init[None, :]
    )

    p0, od = _s1_call(apre, e_pad, src, dst, wbot)

    node_sh = jax.ShapeDtypeStruct((NPAD, HID), f32)
    agg1, c1, cs = _tc_call(_t2_body, (node_sh, node_sh,
                                       jax.ShapeDtypeStruct((1, HID), f32)))(
        p0, W_layers[0], b_layers[0][None, :]
    )
    p1 = _s2_call(c1, src, dst)
    agg2, c2 = _tc_call(_t3_body, (node_sh, node_sh))(
        agg1, p1, W_layers[1], b_layers[1][None, :]
    )
    p2 = _s2_call(c2, src, dst)
    out = _tc_call(_t4_body, jax.ShapeDtypeStruct((1, 1), f32))(
        agg2, p2, W_layers[2], b_layers[2][None, :], od, cs, c1, c2,
        W_ro, W_pred, b_pred[None, :]
    )
    return out
